# Initial kernel scaffold; baseline (speedup 1.0000x reference)
import jax
import jax.numpy as jnp
from jax.experimental import pallas as pl


def kernel(tokens_idx, emb_table):
    raise NotImplementedError("write your pallas kernel here")


# speedup vs baseline: 6.1388x; 6.1388x over previous
import functools
import math

import jax
import jax.numpy as jnp
from jax import lax
from jax.experimental import pallas as pl
from jax.experimental.pallas import tpu as pltpu
from jax.experimental.pallas import tpu_sc as plsc

VOCAB = 128
EMB = 128
B = 4096
S = 200
PAD = 0

NC = 2
NS = 16
NW = NC * NS
SEQ_PER_W = B // NW
ROWS_PER_W = SEQ_PER_W * S
GROUP = 128
NGROUP = ROWS_PER_W // GROUP


def _build_body(tokens_ref, table_ref, f_ref, mask_ref, idx_ref):
    tab = table_ref[...]
    row_ids = lax.broadcasted_iota(jnp.int32, (VOCAB, EMB), 0)
    tab = jnp.where(row_ids == PAD, 0.0, tab)

    pos = lax.broadcasted_iota(jnp.int32, (S, EMB), 0).astype(jnp.float32)
    d = lax.broadcasted_iota(jnp.int32, (S, EMB), 1)
    dt = jnp.exp(((d // 2) * 2).astype(jnp.float32) * (-math.log(10000.0) / EMB))
    ang = pos * dt
    pe = jnp.where(d % 2 == 0, jnp.sin(ang), jnp.cos(ang))

    f_ref[...] = tab[None, :, :] + pe[:, None, :]

    tok = tokens_ref[...]
    mask_ref[...] = tok == PAD
    s_ids = lax.broadcasted_iota(jnp.int32, (B, S), 1)
    idx_ref[...] = s_ids * VOCAB + tok


def _build(tokens_idx, emb_table):
    return pl.pallas_call(
        _build_body,
        out_shape=[
            jax.ShapeDtypeStruct((S, VOCAB, EMB), jnp.float32),
            jax.ShapeDtypeStruct((B, S), jnp.bool_),
            jax.ShapeDtypeStruct((B, S), jnp.int32),
        ],
    )(tokens_idx, emb_table)


_sc_mesh = plsc.VectorSubcoreMesh(core_axis_name="c", subcore_axis_name="s")


@functools.partial(
    pl.kernel,
    out_type=jax.ShapeDtypeStruct((B * S, EMB), jnp.float32),
    mesh=_sc_mesh,
    scratch_types=[
        pltpu.VMEM((NGROUP, GROUP), jnp.int32),
        pltpu.VMEM((GROUP, EMB), jnp.float32),
        pltpu.SemaphoreType.DMA,
    ],
)
def _sc_gather(f_hbm, idx_hbm, out_hbm, idx_v, rows_v, sem):
    wid = lax.axis_index("s") * NC + lax.axis_index("c")
    pltpu.sync_copy(idx_hbm.at[wid], idx_v)
    base = wid * ROWS_PER_W

    @pl.loop(0, NGROUP)
    def _(g):
        pltpu.async_copy(f_hbm.at[idx_v.at[g]], rows_v, sem).wait()
        pltpu.sync_copy(rows_v, out_hbm.at[pl.ds(base + g * GROUP, GROUP)])


def kernel(tokens_idx, emb_table):
    f, mask, idx = _build(tokens_idx, emb_table)
    f_flat = f.reshape(S * VOCAB, EMB)
    idx3 = idx.reshape(NW, NGROUP, GROUP)
    out = _sc_gather(f_flat, idx3)
    return out.reshape(B, S, EMB), mask


# speedup vs baseline: 8.8175x; 1.4364x over previous
import functools
import math

import jax
import jax.numpy as jnp
from jax import lax
from jax.experimental import pallas as pl
from jax.experimental.pallas import tpu as pltpu
from jax.experimental.pallas import tpu_sc as plsc

VOCAB = 128
EMB = 128
B = 4096
S = 200
PAD = 0

NC = 2
NS = 16
NW = NC * NS
SEQ_PER_W = B // NW
ROWS_PER_W = SEQ_PER_W * S
GROUP = 128
NGROUP = ROWS_PER_W // GROUP


def _build_body(tokens_ref, table_ref, f_ref, mask_ref, idx_ref):
    tab = table_ref[...]
    row_ids = lax.broadcasted_iota(jnp.int32, (VOCAB, EMB), 0)
    tab = jnp.where(row_ids == PAD, 0.0, tab)

    pos = lax.broadcasted_iota(jnp.int32, (S, EMB), 0).astype(jnp.float32)
    d = lax.broadcasted_iota(jnp.int32, (S, EMB), 1)
    dt = jnp.exp(((d // 2) * 2).astype(jnp.float32) * (-math.log(10000.0) / EMB))
    ang = pos * dt
    pe = jnp.where(d % 2 == 0, jnp.sin(ang), jnp.cos(ang))

    f_ref[...] = tab[None, :, :] + pe[:, None, :]

    tok = tokens_ref[...]
    mask_ref[...] = tok == PAD
    s_ids = lax.broadcasted_iota(jnp.int32, (B, S), 1)
    idx_ref[...] = s_ids * VOCAB + tok


def _build(tokens_idx, emb_table):
    return pl.pallas_call(
        _build_body,
        out_shape=[
            jax.ShapeDtypeStruct((S, VOCAB, EMB), jnp.float32),
            jax.ShapeDtypeStruct((B, S), jnp.bool_),
            jax.ShapeDtypeStruct((B, S), jnp.int32),
        ],
    )(tokens_idx, emb_table)


_sc_mesh = plsc.VectorSubcoreMesh(core_axis_name="c", subcore_axis_name="s")


NBUF = 4


@functools.partial(
    pl.kernel,
    out_type=jax.ShapeDtypeStruct((B * S, EMB), jnp.float32),
    mesh=_sc_mesh,
    scratch_types=[
        pltpu.VMEM((NGROUP, GROUP), jnp.int32),
        pltpu.VMEM((NBUF, GROUP, EMB), jnp.float32),
    ]
    + [pltpu.SemaphoreType.DMA] * (2 * NBUF),
)
def _sc_gather(f_hbm, idx_hbm, out_hbm, idx_v, bufs, *sems):
    gsems, ssems = sems[:NBUF], sems[NBUF:]
    wid = lax.axis_index("s") * NC + lax.axis_index("c")
    pltpu.sync_copy(idx_hbm.at[wid], idx_v)
    base = wid * ROWS_PER_W

    def gather_start(slot, g):
        pltpu.async_copy(f_hbm.at[idx_v.at[g]], bufs.at[slot], gsems[slot])

    def gather_wait(slot):
        pltpu.make_async_copy(f_hbm.at[idx_v.at[0]], bufs.at[slot], gsems[slot]).wait()

    def scatter_start(slot, g):
        pltpu.async_copy(
            bufs.at[slot], out_hbm.at[pl.ds(base + g * GROUP, GROUP)], ssems[slot]
        )

    def scatter_wait(slot):
        pltpu.make_async_copy(
            bufs.at[slot], out_hbm.at[pl.ds(base, GROUP)], ssems[slot]
        ).wait()

    for b in range(NBUF):
        gather_start(b, b)

    @pl.loop(0, NGROUP // NBUF)
    def _(o):
        for b in range(NBUF):
            g = o * NBUF + b
            gather_wait(b)
            scatter_start(b, g)
            nxt = g + NBUF

            @pl.when(nxt < NGROUP)
            def _():
                scatter_wait(b)
                gather_start(b, nxt)

    for b in range(NBUF):
        scatter_wait(b)


def kernel(tokens_idx, emb_table):
    f, mask, idx = _build(tokens_idx, emb_table)
    f_flat = f.reshape(S * VOCAB, EMB)
    idx3 = idx.reshape(NW, NGROUP, GROUP)
    out = _sc_gather(f_flat, idx3)
    return out.reshape(B, S, EMB), mask
